# Initial kernel scaffold; baseline (speedup 1.0000x reference)
import jax
import jax.numpy as jnp
from jax.experimental import pallas as pl


def kernel(x, edge_index, W1, W_mu, W_logvar):
    raise NotImplementedError("write your pallas kernel here")


# speedup vs baseline: 8.9038x; 8.9038x over previous
import functools

import jax
import jax.numpy as jnp
from jax import lax
from jax.experimental import pallas as pl
from jax.experimental.pallas import tpu as pltpu
from jax.experimental.pallas import tpu_sc as plsc

N = 10000
E = 320000
D = 128
DZ = 64
NC, NS = 2, 16
NW = NC * NS
EPW = E // NW
CH = 80
NCHUNK = EPW // CH
RPS = 624
NREM = N - RPS * NS
ZR = 78

_sc_mesh = plsc.VectorSubcoreMesh(core_axis_name="c", subcore_axis_name="s")


@functools.partial(
    pl.kernel,
    out_type=jax.ShapeDtypeStruct((NC, 2, N, 16), jnp.float32),
    mesh=_sc_mesh,
    compiler_params=pltpu.CompilerParams(use_tc_tiling_on_sc=False),
    scratch_types=[
        pltpu.VMEM((CH,), jnp.int32),
        pltpu.VMEM((CH,), jnp.int32),
        pltpu.VMEM((CH, 16), jnp.float32),
        pltpu.VMEM((ZR, 16), jnp.float32),
        pltpu.VMEM_SHARED((N, 16), jnp.float32),
        pltpu.VMEM_SHARED((N, 16), jnp.float32),
    ],
)
def _deg_kernel(src_hbm, dst_hbm, out_hbm, idx_s, idx_d, ones_v, zeros_v,
                acc_s, acc_d):
    cid = lax.axis_index("c")
    sid = lax.axis_index("s")
    wid = cid * NS + sid

    def fill(i, _):
        ones_v[i, :] = jnp.ones((16,), jnp.float32)
        return 0
    lax.fori_loop(0, CH, fill, 0)

    def zfill(i, _):
        zeros_v[i, :] = jnp.zeros((16,), jnp.float32)
        return 0
    lax.fori_loop(0, ZR, zfill, 0)

    for acc in (acc_s, acc_d):
        for i in range(RPS // ZR):
            pltpu.sync_copy(zeros_v, acc.at[pl.ds(sid * RPS + i * ZR, ZR)])
        @pl.when(sid == NS - 1)
        def _(acc=acc):
            pltpu.sync_copy(zeros_v.at[pl.ds(0, NREM)],
                            acc.at[pl.ds(RPS * NS, NREM)])
    plsc.subcore_barrier()

    def body(c, _):
        eb = wid * EPW + c * CH
        pltpu.sync_copy(src_hbm.at[pl.ds(eb, CH)], idx_s)
        pltpu.sync_copy(dst_hbm.at[pl.ds(eb, CH)], idx_d)
        pltpu.sync_copy(ones_v, acc_s.at[idx_s], add=True)
        pltpu.sync_copy(ones_v, acc_d.at[idx_d], add=True)
        return 0
    lax.fori_loop(0, NCHUNK, body, 0)
    plsc.subcore_barrier()

    sl = pl.ds(sid * RPS, RPS)
    pltpu.sync_copy(acc_s.at[sl], out_hbm.at[cid, 0, sl])
    pltpu.sync_copy(acc_d.at[sl], out_hbm.at[cid, 1, sl])
    @pl.when(sid == NS - 1)
    def _():
        rem = pl.ds(RPS * NS, NREM)
        pltpu.sync_copy(acc_s.at[rem], out_hbm.at[cid, 0, rem])
        pltpu.sync_copy(acc_d.at[rem], out_hbm.at[cid, 1, rem])


@functools.partial(
    pl.kernel,
    out_type=jax.ShapeDtypeStruct((NC, N, D), jnp.float32),
    mesh=_sc_mesh,
    scratch_types=[
        pltpu.VMEM((CH,), jnp.int32),
        pltpu.VMEM((CH,), jnp.int32),
        pltpu.VMEM((CH, D), jnp.float32),
        pltpu.VMEM((ZR, D), jnp.float32),
        pltpu.VMEM_SHARED((N, D), jnp.float32),
        pltpu.SemaphoreType.DMA,
    ],
)
def _agg_kernel(y_hbm, src_hbm, dst_hbm, out_hbm, idx_s, idx_d, rows,
                zeros_v, acc, sem):
    cid = lax.axis_index("c")
    sid = lax.axis_index("s")
    wid = cid * NS + sid

    def zfill(i, _):
        for j in range(D // 16):
            zeros_v[i, pl.ds(j * 16, 16)] = jnp.zeros((16,), jnp.float32)
        return 0
    lax.fori_loop(0, ZR, zfill, 0)

    for i in range(RPS // ZR):
        pltpu.sync_copy(zeros_v, acc.at[pl.ds(sid * RPS + i * ZR, ZR)])
    @pl.when(sid == NS - 1)
    def _():
        pltpu.sync_copy(zeros_v.at[pl.ds(0, NREM)],
                        acc.at[pl.ds(RPS * NS, NREM)])
    plsc.subcore_barrier()

    def body(c, _):
        eb = wid * EPW + c * CH
        pltpu.sync_copy(src_hbm.at[pl.ds(eb, CH)], idx_s)
        pltpu.sync_copy(dst_hbm.at[pl.ds(eb, CH)], idx_d)
        pltpu.async_copy(y_hbm.at[idx_s], rows, sem).wait()
        pltpu.sync_copy(rows, acc.at[idx_d], add=True)
        return 0
    lax.fori_loop(0, NCHUNK, body, 0)
    plsc.subcore_barrier()

    sl = pl.ds(sid * RPS, RPS)
    pltpu.sync_copy(acc.at[sl], out_hbm.at[cid, sl])
    @pl.when(sid == NS - 1)
    def _():
        rem = pl.ds(RPS * NS, NREM)
        pltpu.sync_copy(acc.at[rem], out_hbm.at[cid, rem])


_RB = 1000
_GRID = N // _RB

_deg_spec = pl.BlockSpec((NC, 2, _RB, 16), lambda r: (0, 0, r, 0))


def _norms(degref):
    deg_out = degref[0, 0, :, :1] + degref[1, 0, :, :1]
    deg_in = degref[0, 1, :, :1] + degref[1, 1, :, :1]
    n_out = lax.rsqrt(jnp.maximum(deg_out, 1.0))
    n_in = lax.rsqrt(jnp.maximum(deg_in, 1.0))
    return n_out, n_in


def _tc1_body(x_ref, w_ref, deg_ref, y_ref):
    n_out, _ = _norms(deg_ref)
    xw = jnp.dot(x_ref[...], w_ref[...], preferred_element_type=jnp.float32)
    y_ref[...] = xw * n_out


_tc1 = pl.pallas_call(
    _tc1_body,
    grid=(_GRID,),
    in_specs=[
        pl.BlockSpec((_RB, D), lambda r: (r, 0)),
        pl.BlockSpec((D, D), lambda r: (0, 0)),
        _deg_spec,
    ],
    out_specs=pl.BlockSpec((_RB, D), lambda r: (r, 0)),
    out_shape=jax.ShapeDtypeStruct((N, D), jnp.float32),
)


def _tc2_body(p_ref, deg_ref, y_ref):
    n_out, n_in = _norms(deg_ref)
    h = jnp.maximum((p_ref[0] + p_ref[1]) * n_in, 0.0)
    y_ref[...] = h * n_out


_tc2 = pl.pallas_call(
    _tc2_body,
    grid=(_GRID,),
    in_specs=[
        pl.BlockSpec((NC, _RB, D), lambda r: (0, r, 0)),
        _deg_spec,
    ],
    out_specs=pl.BlockSpec((_RB, D), lambda r: (r, 0)),
    out_shape=jax.ShapeDtypeStruct((N, D), jnp.float32),
)


def _tc3_body(p_ref, deg_ref, wmu_ref, wlv_ref, eps_ref, z_ref, mu_ref,
              lv_ref):
    _, n_in = _norms(deg_ref)
    agg = (p_ref[0] + p_ref[1]) * n_in
    mu = jnp.dot(agg, wmu_ref[...], preferred_element_type=jnp.float32)
    lv = jnp.dot(agg, wlv_ref[...], preferred_element_type=jnp.float32)
    mu_ref[...] = mu
    lv_ref[...] = lv
    z_ref[...] = mu + jnp.exp(0.5 * lv) * eps_ref[...]


_tc3 = pl.pallas_call(
    _tc3_body,
    grid=(_GRID,),
    in_specs=[
        pl.BlockSpec((NC, _RB, D), lambda r: (0, r, 0)),
        _deg_spec,
        pl.BlockSpec((D, DZ), lambda r: (0, 0)),
        pl.BlockSpec((D, DZ), lambda r: (0, 0)),
        pl.BlockSpec((_RB, DZ), lambda r: (r, 0)),
    ],
    out_specs=[
        pl.BlockSpec((_RB, DZ), lambda r: (r, 0)),
        pl.BlockSpec((_RB, DZ), lambda r: (r, 0)),
        pl.BlockSpec((_RB, DZ), lambda r: (r, 0)),
    ],
    out_shape=[
        jax.ShapeDtypeStruct((N, DZ), jnp.float32),
        jax.ShapeDtypeStruct((N, DZ), jnp.float32),
        jax.ShapeDtypeStruct((N, DZ), jnp.float32),
    ],
)


def kernel(x, edge_index, W1, W_mu, W_logvar):
    src = edge_index[0]
    dst = edge_index[1]
    degs = _deg_kernel(src, dst)
    y1 = _tc1(x, W1, degs)
    p1 = _agg_kernel(y1, src, dst)
    y2 = _tc2(p1, degs)
    p2 = _agg_kernel(y2, src, dst)
    eps = jax.random.normal(jax.random.key(42), (N, DZ), dtype=jnp.float32)
    z, mu, lv = _tc3(p2, degs, W_mu, W_logvar, eps)
    return z, mu, lv
